# baseline (speedup 1.0000x reference)
import functools

import jax
import jax.numpy as jnp
from jax import lax
from jax.experimental import pallas as pl
from jax.experimental.pallas import tpu as pltpu
from jax.experimental.pallas import tpu_sc as plsc

EMBED = 128
BATCH = 4096
SEQ = 200
ROWS = BATCH * SEQ
NC, NS = 2, 16
NW = NC * NS
BPW = BATCH // NW
LANES = 16
NBLK = EMBED // LANES
CB = 2
CH = CB * SEQ
NCHUNK = BPW // CB
Q = SEQ // 8


def _body(maskT_hbm, fw_hbm, vr_hbm, out_hbm, fw_v, vr_v, mask_v,
          rows_v0, rows_v1, msem, osem0, osem1):
    cid = lax.axis_index("c")
    sid = lax.axis_index("s")
    wid = sid * NC + cid
    col0 = wid * BPW
    rbase = col0 * SEQ
    rows_v = (rows_v0, rows_v1)
    osem = (osem0, osem1)

    pltpu.sync_copy(fw_hbm, fw_v)
    pltpu.sync_copy(vr_hbm, vr_v)
    fw = [fw_v[pl.ds(c * LANES, LANES)] for c in range(NBLK)]
    vr = [vr_v[pl.ds(c * LANES, LANES)] for c in range(NBLK)]
    df = [vr[c] - fw[c] for c in range(NBLK)]

    dnums = lax.GatherDimensionNumbers(
        offset_dims=(), collapsed_slice_dims=(0,), start_index_map=(0,)
    )
    bidx = [jnp.full((LANES, 1), r, jnp.int32) for r in range(LANES)]

    mcopies = [
        pltpu.make_async_copy(
            maskT_hbm.at[pl.ds(8 * q, 8), pl.ds(col0, BPW)],
            mask_v.at[pl.ds(8 * q, 8)],
            msem,
        )
        for q in range(Q)
    ]
    for cp in mcopies:
        cp.start()
    for cp in mcopies:
        cp.wait()

    @pl.loop(0, NCHUNK // 8)
    def _kg(kg):
        for c8 in range(8):
            b = c8 % 2
            off = rbase + (kg * 8 + c8) * CH

            @pl.when(kg * 8 + c8 >= 2)
            def _():
                pltpu.make_async_copy(
                    rows_v[b], out_hbm.at[pl.ds(off, CH)], osem[b]
                ).wait()

            @pl.loop(0, Q)
            def _q(q):
                for si in range(8):
                    s = 8 * q + si
                    m16 = mask_v[s, pl.ds(LANES * kg, LANES)]
                    mf = m16.astype(jnp.float32)
                    for bi in range(CB):
                        m = lax.gather(
                            mf, bidx[2 * c8 + bi], dnums, (1,),
                            mode=lax.GatherScatterMode.PROMISE_IN_BOUNDS,
                        )
                        for cb in range(NBLK):
                            rows_v[b][bi * SEQ + s, pl.ds(cb * LANES, LANES)] = (
                                fw[cb] + m * df[cb]
                            )

            pltpu.async_copy(rows_v[b], out_hbm.at[pl.ds(off, CH)], osem[b])

    for b in range(2):
        off = rbase + (NCHUNK - 2 + b) * CH
        pltpu.make_async_copy(
            rows_v[b], out_hbm.at[pl.ds(off, CH)], osem[b]
        ).wait()


_sc_call = pl.kernel(
    _body,
    out_type=jax.ShapeDtypeStruct((ROWS, EMBED), jnp.float32),
    mesh=plsc.VectorSubcoreMesh(
        core_axis_name="c", subcore_axis_name="s", num_cores=NC, num_subcores=NS
    ),
    scratch_types=[
        pltpu.VMEM((EMBED,), jnp.float32),
        pltpu.VMEM((EMBED,), jnp.float32),
        pltpu.VMEM((SEQ, BPW), jnp.int32),
        pltpu.VMEM((CH, EMBED), jnp.float32),
        pltpu.VMEM((CH, EMBED), jnp.float32),
        pltpu.SemaphoreType.DMA,
        pltpu.SemaphoreType.DMA,
        pltpu.SemaphoreType.DMA,
    ],
)


@jax.jit
def kernel(position_mask, framework_emb, variable_emb):
    mask_t = position_mask.T
    out = _sc_call(mask_t, framework_emb, variable_emb)
    return out.reshape(BATCH, SEQ, EMBED)


# speedup vs baseline: 1.3821x; 1.3821x over previous
import functools

import jax
import jax.numpy as jnp
from jax import lax
from jax.experimental import pallas as pl
from jax.experimental.pallas import tpu as pltpu
from jax.experimental.pallas import tpu_sc as plsc

EMBED = 128
BATCH = 4096
SEQ = 200
ROWS = BATCH * SEQ
NC, NS = 2, 16
NW = NC * NS
BPW = BATCH // NW
LANES = 16
NBLK = EMBED // LANES
CBAT = 8
CSEQ = 40
NBG = BPW // CBAT
NSC = SEQ // CSEQ
Q = SEQ // 8


def _body(maskT_hbm, fw_hbm, vr_hbm, out_hbm, fw_v, vr_v, mask_v,
          rows_v0, rows_v1, msem, osem0, osem1):
    cid = lax.axis_index("c")
    sid = lax.axis_index("s")
    wid = sid * NC + cid
    col0 = wid * BPW
    rows_v = (rows_v0, rows_v1)
    osem = (osem0, osem1)

    pltpu.sync_copy(fw_hbm, fw_v)
    pltpu.sync_copy(vr_hbm, vr_v)
    fw = [fw_v[pl.ds(c * LANES, LANES)] for c in range(NBLK)]
    vr = [vr_v[pl.ds(c * LANES, LANES)] for c in range(NBLK)]
    df = [vr[c] - fw[c] for c in range(NBLK)]

    dnums = lax.GatherDimensionNumbers(
        offset_dims=(), collapsed_slice_dims=(0,), start_index_map=(0,)
    )
    bidx = [jnp.full((LANES, 1), r, jnp.int32) for r in range(LANES)]

    mcopies = [
        pltpu.make_async_copy(
            maskT_hbm.at[pl.ds(8 * q, 8), pl.ds(col0, BPW)],
            mask_v.at[pl.ds(8 * q, 8)],
            msem,
        )
        for q in range(Q)
    ]
    for cp in mcopies:
        cp.start()
    for cp in mcopies:
        cp.wait()

    def _out_copies(b, bg8, sc):
        cps = []
        for r in range(CBAT):
            off = (col0 + CBAT * bg8 + r) * SEQ + CSEQ * sc
            cps.append(pltpu.make_async_copy(
                rows_v[b].at[r], out_hbm.at[pl.ds(off, CSEQ)], osem[b]
            ))
        return cps

    @pl.loop(0, NBG // 2)
    def _bgp(bgp):
        for half in range(2):
            bg8 = 2 * bgp + half
            lane0 = CBAT * half
            for sc in range(NSC):
                b = (half + sc) % 2
                k = bg8 * NSC + sc

                @pl.when(k >= 2)
                def _():
                    for cp in _out_copies(b, bg8, sc):
                        cp.wait()

                @pl.loop(0, CSEQ)
                def _sl(sl):
                    s = sc * CSEQ + sl
                    m16 = mask_v[s, pl.ds(LANES * bgp, LANES)]
                    mf = m16.astype(jnp.float32)
                    for r in range(CBAT):
                        m = lax.gather(
                            mf, bidx[lane0 + r], dnums, (1,),
                            mode=lax.GatherScatterMode.PROMISE_IN_BOUNDS,
                        )
                        for cb in range(NBLK):
                            rows_v[b][r, sl, pl.ds(cb * LANES, LANES)] = (
                                fw[cb] + m * df[cb]
                            )

                for cp in _out_copies(b, bg8, sc):
                    cp.start()

    for sc, b in ((NSC - 2, 0), (NSC - 1, 1)):
        for r in range(CBAT):
            off = (col0 + CBAT * (NBG - 1) + r) * SEQ + CSEQ * sc
            pltpu.make_async_copy(
                rows_v[b].at[r], out_hbm.at[pl.ds(off, CSEQ)], osem[b]
            ).wait()


_sc_call = pl.kernel(
    _body,
    out_type=jax.ShapeDtypeStruct((ROWS, EMBED), jnp.float32),
    mesh=plsc.VectorSubcoreMesh(
        core_axis_name="c", subcore_axis_name="s", num_cores=NC, num_subcores=NS
    ),
    scratch_types=[
        pltpu.VMEM((EMBED,), jnp.float32),
        pltpu.VMEM((EMBED,), jnp.float32),
        pltpu.VMEM((SEQ, BPW), jnp.int32),
        pltpu.VMEM((CBAT, CSEQ, EMBED), jnp.float32),
        pltpu.VMEM((CBAT, CSEQ, EMBED), jnp.float32),
        pltpu.SemaphoreType.DMA,
        pltpu.SemaphoreType.DMA,
        pltpu.SemaphoreType.DMA,
    ],
)


@jax.jit
def kernel(position_mask, framework_emb, variable_emb):
    mask_t = position_mask.T
    out = _sc_call(mask_t, framework_emb, variable_emb)
    return out.reshape(BATCH, SEQ, EMBED)


# speedup vs baseline: 1.4574x; 1.0544x over previous
import functools

import jax
import jax.numpy as jnp
from jax import lax
from jax.experimental import pallas as pl
from jax.experimental.pallas import tpu as pltpu
from jax.experimental.pallas import tpu_sc as plsc

EMBED = 128
BATCH = 4096
SEQ = 200
ROWS = BATCH * SEQ
NC, NS = 2, 16
NW = NC * NS
ROWS_PER_W = ROWS // NW
CH = 256
NCHUNK = ROWS_PER_W // CH
NCH0 = 98
NCH1 = 102
LANES = 16
NBLK = EMBED // LANES


def _body(mask_hbm, fw_hbm, vr_hbm, out_hbm, fw_v, vr_v,
          mask_v0, mask_v1, rows_v0, rows_v1,
          msem0, msem1, osem0, osem1):
    cid = lax.axis_index("c")
    sid = lax.axis_index("s")
    nchunk = NCH0 + cid * (NCH1 - NCH0)
    base = (sid * (NCH0 + NCH1) + cid * NCH0) * CH
    mask_v = (mask_v0, mask_v1)
    rows_v = (rows_v0, rows_v1)
    msem = (msem0, msem1)
    osem = (osem0, osem1)

    pltpu.sync_copy(fw_hbm, fw_v)
    pltpu.sync_copy(vr_hbm, vr_v)
    fw = [fw_v[pl.ds(c * LANES, LANES)] for c in range(NBLK)]
    vr = [vr_v[pl.ds(c * LANES, LANES)] for c in range(NBLK)]
    df = [vr[c] - fw[c] for c in range(NBLK)]

    dnums = lax.GatherDimensionNumbers(
        offset_dims=(), collapsed_slice_dims=(0,), start_index_map=(0,)
    )
    bidx = [jnp.full((LANES, 1), r, jnp.int32) for r in range(LANES)]

    for b in range(2):
        pltpu.async_copy(mask_hbm.at[pl.ds(base + b * CH, CH)], mask_v[b], msem[b])

    @pl.loop(0, nchunk, step=2)
    def _pair(i):
        for b in range(2):
            c = i + b
            off = base + c * CH
            pltpu.make_async_copy(
                mask_hbm.at[pl.ds(off, CH)], mask_v[b], msem[b]
            ).wait()

            @pl.when(c >= 2)
            def _():
                pltpu.make_async_copy(
                    rows_v[b], out_hbm.at[pl.ds(off, CH)], osem[b]
                ).wait()

            @pl.loop(0, CH // LANES)
            def _group(g):
                m16 = mask_v[b][pl.ds(g * LANES, LANES)]
                for r in range(LANES):
                    m = lax.gather(
                        m16, bidx[r], dnums, (1,),
                        mode=lax.GatherScatterMode.PROMISE_IN_BOUNDS,
                    ).astype(jnp.float32)
                    row = g * LANES + r
                    for cb in range(NBLK):
                        rows_v[b][row, pl.ds(cb * LANES, LANES)] = (
                            fw[cb] + m * df[cb]
                        )

            pltpu.async_copy(rows_v[b], out_hbm.at[pl.ds(off, CH)], osem[b])

            @pl.when(c + 2 < nchunk)
            def _():
                pltpu.async_copy(
                    mask_hbm.at[pl.ds(base + (c + 2) * CH, CH)],
                    mask_v[b], msem[b],
                )

    for b in range(2):
        off = base + (nchunk - 2 + b) * CH
        pltpu.make_async_copy(
            rows_v[b], out_hbm.at[pl.ds(off, CH)], osem[b]
        ).wait()


_sc_call = pl.kernel(
    _body,
    out_type=jax.ShapeDtypeStruct((ROWS, EMBED), jnp.float32),
    mesh=plsc.VectorSubcoreMesh(
        core_axis_name="c", subcore_axis_name="s", num_cores=NC, num_subcores=NS
    ),
    scratch_types=[
        pltpu.VMEM((EMBED,), jnp.float32),
        pltpu.VMEM((EMBED,), jnp.float32),
        pltpu.VMEM((CH,), jnp.int32),
        pltpu.VMEM((CH,), jnp.int32),
        pltpu.VMEM((CH, EMBED), jnp.float32),
        pltpu.VMEM((CH, EMBED), jnp.float32),
        pltpu.SemaphoreType.DMA,
        pltpu.SemaphoreType.DMA,
        pltpu.SemaphoreType.DMA,
        pltpu.SemaphoreType.DMA,
    ],
)


@jax.jit
def kernel(position_mask, framework_emb, variable_emb):
    mask_flat = position_mask.reshape(ROWS)
    out = _sc_call(mask_flat, framework_emb, variable_emb)
    return out.reshape(BATCH, SEQ, EMBED)
